# baseline (speedup 1.0000x reference)
import functools

import jax
import jax.numpy as jnp
from jax import lax
from jax.experimental import pallas as pl
from jax.experimental.pallas import tpu as pltpu
from jax.experimental.pallas import tpu_sc as plsc

_CB = 256
_C = 256


def _make_repack(V, D, nc, ns):
    NW = nc * ns
    CB = _CB
    nfull = V // CB
    rem = V - nfull * CB
    assert rem % 16 == 0
    mesh = plsc.VectorSubcoreMesh(core_axis_name="c", subcore_axis_name="s")

    @functools.partial(
        pl.kernel,
        out_type=jax.ShapeDtypeStruct((V // 2, 2 * D), jnp.float32),
        mesh=mesh,
        scratch_types=[
            pltpu.VMEM((D, CB), jnp.float32),
            pltpu.VMEM((D, CB), jnp.float32),
            pltpu.VMEM((CB // 2, 2 * D), jnp.float32),
            pltpu.VMEM((CB // 2, 2 * D), jnp.float32),
            pltpu.SemaphoreType.DMA,
            pltpu.SemaphoreType.DMA,
            pltpu.SemaphoreType.DMA,
            pltpu.SemaphoreType.DMA,
        ],
        compiler_params=pltpu.CompilerParams(needs_layout_passes=False),
    )
    def repack(wt_hbm, wtail_hbm, w2_hbm, g0, g1, t0, t1, gs0, gs1, ts0, ts1):
        wid = lax.axis_index("s") * nc + lax.axis_index("c")
        gbufs, tbufs = (g0, g1), (t0, t1)
        gsems, tsems = (gs0, gs1), (ts0, ts1)

        def read(s, blk):
            return pltpu.make_async_copy(
                wt_hbm.at[:, pl.ds(pl.multiple_of(blk * CB, CB), CB)],
                gbufs[s], gsems[s])

        def write(s, blk):
            return pltpu.make_async_copy(
                tbufs[s],
                w2_hbm.at[pl.ds(pl.multiple_of(blk * (CB // 2), CB // 2),
                                CB // 2), :], tsems[s])

        cnt = (nfull - wid + NW - 1) // NW

        lanes = jax.lax.iota(jnp.int32, 16)
        ccjs = [(lanes + j) & 15 for j in range(16)]
        prow = lanes >> 1
        pccjs = [((lanes & 1) * D) + cc for cc in ccjs]

        def transpose_blk(s):
            @plsc.parallel_loop(0, CB // 16, unroll=4)
            def c_body(cg):
                c0 = cg * 16
                gcol = lanes + c0
                wrow = prow + (c0 >> 1)

                @plsc.parallel_loop(0, D // 16, unroll=4)
                def _(m):
                    l0 = m * 16
                    for j in range(16):
                        v = plsc.load_gather(gbufs[s], [ccjs[j] + l0, gcol])
                        plsc.store_scatter(tbufs[s], [wrow, pccjs[j] + l0], v)


        kmax = ((nfull + NW - 1) // NW + 1) // 2

        @pl.when(cnt > 0)
        def _():
            read(0, wid).start()

        def body(k2, _):
            for ks in range(2):
                k = 2 * k2 + ks

                @pl.when(k < cnt)
                def _():
                    blk = wid + k * NW

                    @pl.when(k + 1 < cnt)
                    def _():
                        read(1 - ks, wid + (k + 1) * NW).start()
                    read(ks, blk).wait()

                    @pl.when(k >= 2)
                    def _():
                        write(ks, blk - 2 * NW).wait()
                    transpose_blk(ks)
                    write(ks, blk).start()

            return 0

        lax.fori_loop(0, kmax, body, 0)
        for ks in range(2):
            @pl.when(cnt > ks)
            def _():
                last_k = cnt - 1 - ((cnt - 1 - ks) % 2)
                write(ks, wid + last_k * NW).wait()

        if rem:
            @pl.when(wid == 0)
            def _():
                pltpu.sync_copy(wtail_hbm, tbufs[0].at[pl.ds(0, rem // 2), :])
                pltpu.sync_copy(
                    tbufs[0].at[pl.ds(0, rem // 2), :],
                    w2_hbm.at[pl.ds(nfull * (CB // 2), rem // 2), :])

    return repack


def _make_gather(B, V, D, R, S, nc, ns):
    NW = nc * ns
    b_per_w = B // NW
    C = _C
    nchunks = b_per_w // C
    mesh = plsc.VectorSubcoreMesh(core_axis_name="c", subcore_axis_name="s")

    @functools.partial(
        pl.kernel,
        out_type=jax.ShapeDtypeStruct((R, D, S), jnp.float32),
        mesh=mesh,
        scratch_types=[
            pltpu.VMEM((b_per_w,), jnp.int32),
            pltpu.VMEM((C,), jnp.int32),
            pltpu.VMEM((C,), jnp.int32),
            pltpu.VMEM((C, 2 * D), jnp.float32),
            pltpu.VMEM((C, 2 * D), jnp.float32),
            pltpu.VMEM((D, C), jnp.float32),
            pltpu.VMEM((D, C), jnp.float32),
            pltpu.SemaphoreType.DMA,
            pltpu.SemaphoreType.DMA,
            pltpu.SemaphoreType.DMA,
            pltpu.SemaphoreType.DMA,
        ],
        compiler_params=pltpu.CompilerParams(needs_layout_passes=False),
    )
    def gather(idx_hbm, w2_hbm, out_hbm, idx_v, p0, p1, g0, g1, t0, t1,
               gs0, gs1, ts0, ts1):
        wid = lax.axis_index("s") * nc + lax.axis_index("c")
        base = wid * b_per_w
        pbufs, gbufs, tbufs = (p0, p1), (g0, g1), (t0, t1)
        gsems, tsems = (gs0, gs1), (ts0, ts1)
        pltpu.sync_copy(idx_hbm.at[pl.ds(base, b_per_w)], idx_v)

        def fill_pairs(s, c):
            @plsc.parallel_loop(0, C // 16, unroll=8)
            def _(v):
                pbufs[s][pl.ds(v * 16, 16)] = (
                    idx_v[pl.ds(c * C + v * 16, 16)] >> 1)

        def gstart(s):
            return pltpu.make_async_copy(
                w2_hbm.at[pbufs[s]], gbufs[s], gsems[s])

        def wstart(s, c):
            b0 = base + c * C
            i = b0 // S
            j0 = pl.multiple_of(b0 % S, C)
            return pltpu.make_async_copy(
                tbufs[s], out_hbm.at[i, :, pl.ds(j0, C)], tsems[s])

        lanes = jax.lax.iota(jnp.int32, 16)
        ccjs = [(lanes + j) & 15 for j in range(16)]

        def transpose_chunk(s, c):
            @plsc.parallel_loop(0, C // 16, unroll=4)
            def cg_body(cg):
                c0 = cg * 16
                grow = lanes + c0
                hoff = (idx_v[pl.ds(c * C + c0, 16)] & 1) * D

                @plsc.parallel_loop(0, D // 16, unroll=4)
                def _(m):
                    l0 = m * 16
                    for j in range(16):
                        lrow = ccjs[j] + l0
                        v = plsc.load_gather(gbufs[s], [grow, hoff + lrow])
                        plsc.store_scatter(tbufs[s], [lrow, grow], v)


        assert nchunks % 2 == 0
        fill_pairs(0, 0)
        gstart(0).start()

        def body(c2, _):
            for ss in range(2):
                c = 2 * c2 + ss

                @pl.when(c + 1 < nchunks)
                def _():
                    fill_pairs(1 - ss, c + 1)
                    gstart(1 - ss).start()
                gstart(ss).wait()

                @pl.when(c >= 2)
                def _():
                    wstart(ss, c - 2).wait()
                transpose_chunk(ss, c)
                wstart(ss, c).start()
            return 0

        lax.fori_loop(0, nchunks // 2, body, 0)
        wstart(0, nchunks - 2).wait()
        wstart(1, nchunks - 1).wait()

    return gather


def kernel(input_tensor, weight):
    R, S = input_tensor.shape
    V, D = weight.shape
    B = R * S
    idx_flat = input_tensor.reshape(B).astype(jnp.int32)
    wt = weight.T
    info = plsc.get_sparse_core_info()
    nc, ns = info.num_cores, info.num_subcores
    rem = V % _CB
    wtail = weight[V - rem:, :].reshape(rem // 2, 2 * D)
    w2 = _make_repack(V, D, nc, ns)(wt, wtail)
    out_t = _make_gather(B, V, D, R, S, nc, ns)(idx_flat, w2)
    return jnp.transpose(out_t, (0, 2, 1))


# speedup vs baseline: 1.0443x; 1.0443x over previous
import functools

import jax
import jax.numpy as jnp
from jax import lax
from jax.experimental import pallas as pl
from jax.experimental.pallas import tpu as pltpu
from jax.experimental.pallas import tpu_sc as plsc

_CB = 256
_C = 256


def _make_repack(V, D, nc, ns):
    NW = nc * ns
    CB = _CB
    nfull = V // CB
    rem = V - nfull * CB
    assert rem % 16 == 0
    mesh = plsc.VectorSubcoreMesh(core_axis_name="c", subcore_axis_name="s")

    @functools.partial(
        pl.kernel,
        out_type=jax.ShapeDtypeStruct((V // 2, 2 * D), jnp.float32),
        mesh=mesh,
        scratch_types=[
            pltpu.VMEM((D, CB), jnp.float32),
            pltpu.VMEM((D, CB), jnp.float32),
            pltpu.VMEM((CB // 2, 2 * D), jnp.float32),
            pltpu.VMEM((CB // 2, 2 * D), jnp.float32),
            pltpu.SemaphoreType.DMA,
            pltpu.SemaphoreType.DMA,
            pltpu.SemaphoreType.DMA,
            pltpu.SemaphoreType.DMA,
        ],
        compiler_params=pltpu.CompilerParams(needs_layout_passes=False),
    )
    def repack(wt_hbm, wtail_hbm, w2_hbm, g0, g1, t0, t1, gs0, gs1, ts0, ts1):
        wid = lax.axis_index("s") * nc + lax.axis_index("c")
        gbufs, tbufs = (g0, g1), (t0, t1)
        gsems, tsems = (gs0, gs1), (ts0, ts1)

        def read(s, blk):
            return pltpu.make_async_copy(
                wt_hbm.at[:, pl.ds(pl.multiple_of(blk * CB, CB), CB)],
                gbufs[s], gsems[s])

        def write(s, blk):
            return pltpu.make_async_copy(
                tbufs[s],
                w2_hbm.at[pl.ds(pl.multiple_of(blk * (CB // 2), CB // 2),
                                CB // 2), :], tsems[s])

        cnt = (nfull - wid + NW - 1) // NW

        lanes = jax.lax.iota(jnp.int32, 16)
        ccjs = [(lanes + j) & 15 for j in range(16)]
        prow = lanes >> 1
        pccjs = [((lanes & 1) * D) + cc for cc in ccjs]

        def transpose_blk(s):
            @plsc.parallel_loop(0, CB // 16, unroll=2)
            def c_body(cg):
                c0 = cg * 16
                gcol = lanes + c0
                wrow = prow + (c0 >> 1)

                @plsc.parallel_loop(0, D // 16, unroll=4)
                def _(m):
                    l0 = m * 16
                    for j in range(16):
                        v = plsc.load_gather(gbufs[s], [ccjs[j] + l0, gcol])
                        plsc.store_scatter(tbufs[s], [wrow, pccjs[j] + l0], v)


        kmax = ((nfull + NW - 1) // NW + 1) // 2

        @pl.when(cnt > 0)
        def _():
            read(0, wid).start()

        def body(k2, _):
            for ks in range(2):
                k = 2 * k2 + ks

                @pl.when(k < cnt)
                def _():
                    blk = wid + k * NW

                    @pl.when(k + 1 < cnt)
                    def _():
                        read(1 - ks, wid + (k + 1) * NW).start()
                    read(ks, blk).wait()

                    @pl.when(k >= 2)
                    def _():
                        write(ks, blk - 2 * NW).wait()
                    transpose_blk(ks)
                    write(ks, blk).start()

            return 0

        lax.fori_loop(0, kmax, body, 0)
        for ks in range(2):
            @pl.when(cnt > ks)
            def _():
                last_k = cnt - 1 - ((cnt - 1 - ks) % 2)
                write(ks, wid + last_k * NW).wait()

        if rem:
            @pl.when(wid == 0)
            def _():
                pltpu.sync_copy(wtail_hbm, tbufs[0].at[pl.ds(0, rem // 2), :])
                pltpu.sync_copy(
                    tbufs[0].at[pl.ds(0, rem // 2), :],
                    w2_hbm.at[pl.ds(nfull * (CB // 2), rem // 2), :])

    return repack


def _make_gather(B, V, D, R, S, nc, ns):
    NW = nc * ns
    b_per_w = B // NW
    C = _C
    nchunks = b_per_w // C
    mesh = plsc.VectorSubcoreMesh(core_axis_name="c", subcore_axis_name="s")

    @functools.partial(
        pl.kernel,
        out_type=jax.ShapeDtypeStruct((R, D, S), jnp.float32),
        mesh=mesh,
        scratch_types=[
            pltpu.VMEM((b_per_w,), jnp.int32),
            pltpu.VMEM((C,), jnp.int32),
            pltpu.VMEM((C,), jnp.int32),
            pltpu.VMEM((C, 2 * D), jnp.float32),
            pltpu.VMEM((C, 2 * D), jnp.float32),
            pltpu.VMEM((D, C), jnp.float32),
            pltpu.VMEM((D, C), jnp.float32),
            pltpu.SemaphoreType.DMA,
            pltpu.SemaphoreType.DMA,
            pltpu.SemaphoreType.DMA,
            pltpu.SemaphoreType.DMA,
        ],
        compiler_params=pltpu.CompilerParams(needs_layout_passes=False),
    )
    def gather(idx_hbm, w2_hbm, out_hbm, idx_v, p0, p1, g0, g1, t0, t1,
               gs0, gs1, ts0, ts1):
        wid = lax.axis_index("s") * nc + lax.axis_index("c")
        base = wid * b_per_w
        pbufs, gbufs, tbufs = (p0, p1), (g0, g1), (t0, t1)
        gsems, tsems = (gs0, gs1), (ts0, ts1)
        pltpu.sync_copy(idx_hbm.at[pl.ds(base, b_per_w)], idx_v)

        def fill_pairs(s, c):
            @plsc.parallel_loop(0, C // 16, unroll=8)
            def _(v):
                pbufs[s][pl.ds(v * 16, 16)] = (
                    idx_v[pl.ds(c * C + v * 16, 16)] >> 1)

        def gstart(s):
            return pltpu.make_async_copy(
                w2_hbm.at[pbufs[s]], gbufs[s], gsems[s])

        def wstart(s, c):
            b0 = base + c * C
            i = b0 // S
            j0 = pl.multiple_of(b0 % S, C)
            return pltpu.make_async_copy(
                tbufs[s], out_hbm.at[i, :, pl.ds(j0, C)], tsems[s])

        lanes = jax.lax.iota(jnp.int32, 16)
        ccjs = [(lanes + j) & 15 for j in range(16)]

        def transpose_chunk(s, c):
            @plsc.parallel_loop(0, C // 16, unroll=2)
            def cg_body(cg):
                c0 = cg * 16
                grow = lanes + c0
                hoff = (idx_v[pl.ds(c * C + c0, 16)] & 1) * D

                @plsc.parallel_loop(0, D // 16, unroll=4)
                def _(m):
                    l0 = m * 16
                    for j in range(16):
                        lrow = ccjs[j] + l0
                        v = plsc.load_gather(gbufs[s], [grow, hoff + lrow])
                        plsc.store_scatter(tbufs[s], [lrow, grow], v)


        assert nchunks % 2 == 0
        fill_pairs(0, 0)
        gstart(0).start()

        def body(c2, _):
            for ss in range(2):
                c = 2 * c2 + ss

                @pl.when(c + 1 < nchunks)
                def _():
                    fill_pairs(1 - ss, c + 1)
                    gstart(1 - ss).start()
                gstart(ss).wait()

                @pl.when(c >= 2)
                def _():
                    wstart(ss, c - 2).wait()
                transpose_chunk(ss, c)
                wstart(ss, c).start()
            return 0

        lax.fori_loop(0, nchunks // 2, body, 0)
        wstart(0, nchunks - 2).wait()
        wstart(1, nchunks - 1).wait()

    return gather


def kernel(input_tensor, weight):
    R, S = input_tensor.shape
    V, D = weight.shape
    B = R * S
    idx_flat = input_tensor.reshape(B).astype(jnp.int32)
    wt = weight.T
    info = plsc.get_sparse_core_info()
    nc, ns = info.num_cores, info.num_subcores
    rem = V % _CB
    wtail = weight[V - rem:, :].reshape(rem // 2, 2 * D)
    w2 = _make_repack(V, D, nc, ns)(wt, wtail)
    out_t = _make_gather(B, V, D, R, S, nc, ns)(idx_flat, w2)
    return jnp.transpose(out_t, (0, 2, 1))
